# baseline (speedup 1.0000x reference)
import functools

import jax
import jax.numpy as jnp
from jax import lax
from jax.experimental import pallas as pl
from jax.experimental.pallas import tpu as pltpu
from jax.experimental.pallas import tpu_sc as plsc

N_NODES = 1000000
BATCH = 16384
NUM_CORES = 2
NUM_SUBCORES = 16
NW = NUM_CORES * NUM_SUBCORES
B_PER_W = BATCH // NW
CHUNK = 128
NCHUNK = B_PER_W // CHUNK
PITCH = 8
LANES = 16

_mesh = plsc.VectorSubcoreMesh(core_axis_name="c", subcore_axis_name="s")


@functools.partial(
    pl.kernel,
    mesh=_mesh,
    out_type=[jax.ShapeDtypeStruct((BATCH,), jnp.float32)] * 3,
    scratch_types=[
        pltpu.VMEM((B_PER_W,), jnp.int32),
        pltpu.VMEM((B_PER_W, 1), jnp.float32),
        pltpu.VMEM((B_PER_W, 1), jnp.float32),
        pltpu.VMEM((B_PER_W, 1), jnp.float32),
        pltpu.VMEM((B_PER_W,), jnp.float32),
        pltpu.SemaphoreType.DMA,
    ],
    compiler_params=pltpu.CompilerParams(use_tc_tiling_on_sc=False, needs_layout_passes=False),
)
def _gather3(idx_hbm, mu_hbm, sg_hbm, et_hbm, out_mu, out_sg, out_et,
             idx_v, buf_mu, buf_sg, buf_et, pack_v, sem):
    wid = lax.axis_index("s") * NUM_CORES + lax.axis_index("c")
    base = wid * B_PER_W
    pltpu.sync_copy(idx_hbm.at[pl.ds(base, B_PER_W)], idx_v)
    copies = []
    for tbl, buf in ((mu_hbm, buf_mu), (sg_hbm, buf_sg), (et_hbm, buf_et)):
        for j in range(NCHUNK):
            copies.append(
                pltpu.async_copy(
                    tbl.at[idx_v.at[pl.ds(j * CHUNK, CHUNK)]],
                    buf.at[pl.ds(j * CHUNK, CHUNK)], sem))
    for c in copies:
        c.wait()
    zeros = jnp.zeros((LANES,), jnp.int32)
    for buf, out in ((buf_mu, out_mu), (buf_sg, out_sg), (buf_et, out_et)):
        for k in range(B_PER_W // LANES):
            rows = lax.iota(jnp.int32, LANES) + (k * LANES)
            pack_v[pl.ds(k * LANES, LANES)] = plsc.load_gather(
                buf, [rows, zeros])
        pltpu.sync_copy(pack_v, out.at[pl.ds(base, B_PER_W)])


def kernel(indices, mu_w, sigma_w, eta_w):
    mu, sg, et = _gather3(indices.astype(jnp.int32), mu_w, sigma_w, eta_w)
    return jnp.stack([mu, sg, et], axis=-1)


# speedup vs baseline: 17.1522x; 17.1522x over previous
import functools

import jax
import jax.numpy as jnp
from jax import lax
from jax.experimental import pallas as pl
from jax.experimental.pallas import tpu as pltpu
from jax.experimental.pallas import tpu_sc as plsc

N_NODES = 1000000
BATCH = 16384
NUM_CORES = 2
NUM_SUBCORES = 16
NW = NUM_CORES * NUM_SUBCORES
B_PER_W = BATCH // NW

_mesh = plsc.VectorSubcoreMesh(core_axis_name="c", subcore_axis_name="s")


@functools.partial(
    pl.kernel,
    mesh=_mesh,
    out_type=[jax.ShapeDtypeStruct((BATCH,), jnp.float32)] * 3,
    scratch_types=[
        pltpu.VMEM((B_PER_W,), jnp.int32),
        pltpu.VMEM((B_PER_W,), jnp.float32),
        pltpu.VMEM((B_PER_W,), jnp.float32),
        pltpu.VMEM((B_PER_W,), jnp.float32),
        pltpu.SemaphoreType.DMA,
    ],
)
def _gather3(idx_hbm, mu_hbm, sg_hbm, et_hbm, out_mu, out_sg, out_et,
             idx_v, buf_mu, buf_sg, buf_et, sem):
    wid = lax.axis_index("s") * NUM_CORES + lax.axis_index("c")
    base = wid * B_PER_W
    pltpu.sync_copy(idx_hbm.at[pl.ds(base, B_PER_W)], idx_v)
    copies = [
        pltpu.async_copy(tbl.at[idx_v], buf, sem)
        for tbl, buf in ((mu_hbm, buf_mu), (sg_hbm, buf_sg),
                         (et_hbm, buf_et))
    ]
    for c in copies:
        c.wait()
    pltpu.sync_copy(buf_mu, out_mu.at[pl.ds(base, B_PER_W)])
    pltpu.sync_copy(buf_sg, out_sg.at[pl.ds(base, B_PER_W)])
    pltpu.sync_copy(buf_et, out_et.at[pl.ds(base, B_PER_W)])


def kernel(indices, mu_w, sigma_w, eta_w):
    mu, sg, et = _gather3(indices.astype(jnp.int32),
                          mu_w.reshape(-1), sigma_w.reshape(-1),
                          eta_w.reshape(-1))
    return jnp.stack([mu, sg, et], axis=-1)
